# baseline (speedup 1.0000x reference)
import jax
import jax.numpy as jnp
from jax import lax
from jax.experimental import pallas as pl
from jax.experimental.pallas import tpu as pltpu

B = 64
B2 = 2 * B
EDD = 2048
EDS = 1024
CHUNK = 256
NCHUNK = EDS // CHUNK
BN_EPS = 1e-5
COS_EPS = 1e-8

_DN_T = (((1,), (1,)), ((), ()))
_DN = (((1,), (0,)), ((), ()))


def _fused_kernel(xy_ref, w_ref, gx_ref, bx_ref, gy_ref, by_ref, out_ref):
    scol = lax.broadcasted_iota(jnp.int32, (2, B2), 1)
    srow = lax.broadcasted_iota(jnp.int32, (2, B2), 0)
    sel = jnp.where((scol < B) == (srow == 0), 1.0, 0.0).astype(jnp.float32)
    is_top = lax.broadcasted_iota(jnp.int32, (B2, CHUNK), 0) < B
    lane = lax.broadcasted_iota(jnp.int32, (B2, CHUNK), 1)
    at_block_start = (lane % 4) == 0
    low = jnp.full((B2, CHUNK), -2.0, dtype=jnp.float32)
    ones_col = jnp.ones((CHUNK, 1), dtype=jnp.float32)

    dot = jnp.zeros((B, 1), dtype=jnp.float32)
    nx = jnp.zeros((B, 1), dtype=jnp.float32)
    ny = jnp.zeros((B, 1), dtype=jnp.float32)
    for k in range(NCHUNK):
        cols = pl.ds(k * CHUNK, CHUNK)
        w = w_ref[cols, :]
        h = lax.dot_general(xy_ref[...], w, _DN_T,
                            preferred_element_type=jnp.float32)

        s1 = lax.dot_general(sel, h, _DN,
                             preferred_element_type=jnp.float32)
        s2 = lax.dot_general(sel, h * h, _DN,
                             preferred_element_type=jnp.float32)
        mu = s1 * (1.0 / B)
        var = s2 * (1.0 / B) - mu * mu
        rstd = lax.rsqrt(var + BN_EPS)
        g2 = jnp.concatenate([gx_ref[:, cols], gy_ref[:, cols]], axis=0)
        b2 = jnp.concatenate([bx_ref[:, cols], by_ref[:, cols]], axis=0)
        scale2 = rstd * g2
        shift2 = b2 - mu * scale2
        scale = jnp.where(is_top, scale2[0:1], scale2[1:2])
        shift = jnp.where(is_top, shift2[0:1], shift2[1:2])
        t = jnp.tanh(h * scale + shift)

        a = jnp.maximum(t, pltpu.roll(t, CHUNK - 1, 1))
        bm = jnp.maximum(a, pltpu.roll(a, CHUNK - 2, 1))
        c = jnp.where(at_block_start, bm, low)
        c = jnp.maximum(c, pltpu.roll(c, 1, 1))
        bmax = jnp.maximum(c, pltpu.roll(c, 2, 1))
        m = jnp.where(t == bmax, t, 0.0)

        mx = m[:B]
        my = m[B:]
        dot += lax.dot_general(mx * my, ones_col, _DN,
                               preferred_element_type=jnp.float32)
        nx += lax.dot_general(mx * mx, ones_col, _DN,
                              preferred_element_type=jnp.float32)
        ny += lax.dot_general(my * my, ones_col, _DN,
                              preferred_element_type=jnp.float32)

    nxc = jnp.maximum(jnp.sqrt(nx), COS_EPS)
    nyc = jnp.maximum(jnp.sqrt(ny), COS_EPS)
    out_ref[...] = dot / (nxc * nyc)


def kernel(x, y, W, b, gamma_x, beta_x, gamma_y, beta_y):
    xy = jnp.concatenate([x, y], axis=0)
    row = lambda v: v.reshape(1, EDS)
    out = pl.pallas_call(
        _fused_kernel,
        in_specs=[
            pl.BlockSpec((B2, EDD), lambda: (0, 0)),
            pl.BlockSpec((EDS, EDD), lambda: (0, 0)),
            pl.BlockSpec((1, EDS), lambda: (0, 0)),
            pl.BlockSpec((1, EDS), lambda: (0, 0)),
            pl.BlockSpec((1, EDS), lambda: (0, 0)),
            pl.BlockSpec((1, EDS), lambda: (0, 0)),
        ],
        out_specs=pl.BlockSpec((B, 1), lambda: (0, 0)),
        out_shape=jax.ShapeDtypeStruct((B, 1), jnp.float32),
    )(xy, W, row(gamma_x), row(beta_x), row(gamma_y), row(beta_y))
    return out.reshape(B)


# speedup vs baseline: 1.1724x; 1.1724x over previous
import jax
import jax.numpy as jnp
from jax import lax
from jax.experimental import pallas as pl
from jax.experimental.pallas import tpu as pltpu

B = 64
EDD = 2048
EDS = 1024
BLK = 256
NBLK = EDS // BLK
BN_EPS = 1e-5
COS_EPS = 1e-8

_DN_T = (((1,), (1,)), ((), ()))
_DN = (((1,), (0,)), ((), ()))


def _fused_kernel(x_hbm, y_hbm, w_ref, gx_ref, bx_ref, gy_ref, by_ref,
                  out_ref, xbuf, ybuf, acc_dot, acc_nx, acc_ny, sems):
    j = pl.program_id(0)

    @pl.when(j == 0)
    def _():
        cx = pltpu.make_async_copy(x_hbm, xbuf, sems.at[0])
        cy = pltpu.make_async_copy(y_hbm, ybuf, sems.at[1])
        cx.start()
        cy.start()
        cx.wait()
        cy.wait()

    w = w_ref[...]
    hx = lax.dot_general(xbuf[...], w, _DN_T,
                         preferred_element_type=jnp.float32)
    hy = lax.dot_general(ybuf[...], w, _DN_T,
                         preferred_element_type=jnp.float32)

    ones_row = jnp.ones((1, B), dtype=jnp.float32)
    ones_col = jnp.ones((BLK, 1), dtype=jnp.float32)
    lane = lax.broadcasted_iota(jnp.int32, (B, BLK), 1)
    at_block_start = (lane % 4) == 0
    low = jnp.full((B, BLK), -2.0, dtype=jnp.float32)

    def bn_tanh(hh, g, bb):
        s1 = lax.dot_general(ones_row, hh, _DN,
                             preferred_element_type=jnp.float32)
        s2 = lax.dot_general(ones_row, hh * hh, _DN,
                             preferred_element_type=jnp.float32)
        mu = s1 * (1.0 / B)
        var = s2 * (1.0 / B) - mu * mu
        scale = lax.rsqrt(var + BN_EPS) * g
        shift = bb - mu * scale
        return jnp.tanh(hh * scale + shift)

    def block_mask(hh):
        a = jnp.maximum(hh, pltpu.roll(hh, BLK - 1, 1))
        bm = jnp.maximum(a, pltpu.roll(a, BLK - 2, 1))
        c = jnp.where(at_block_start, bm, low)
        c = jnp.maximum(c, pltpu.roll(c, 1, 1))
        bmax = jnp.maximum(c, pltpu.roll(c, 2, 1))
        return jnp.where(hh == bmax, hh, 0.0)

    mx = block_mask(bn_tanh(hx, gx_ref[...], bx_ref[...]))
    my = block_mask(bn_tanh(hy, gy_ref[...], by_ref[...]))

    p_dot = lax.dot_general(mx * my, ones_col, _DN,
                            preferred_element_type=jnp.float32)
    p_nx = lax.dot_general(mx * mx, ones_col, _DN,
                           preferred_element_type=jnp.float32)
    p_ny = lax.dot_general(my * my, ones_col, _DN,
                           preferred_element_type=jnp.float32)

    @pl.when(j == 0)
    def _():
        acc_dot[...] = p_dot
        acc_nx[...] = p_nx
        acc_ny[...] = p_ny

    @pl.when(j != 0)
    def _():
        acc_dot[...] += p_dot
        acc_nx[...] += p_nx
        acc_ny[...] += p_ny

    @pl.when(j == NBLK - 1)
    def _():
        nxc = jnp.maximum(jnp.sqrt(acc_nx[...]), COS_EPS)
        nyc = jnp.maximum(jnp.sqrt(acc_ny[...]), COS_EPS)
        out_ref[...] = acc_dot[...] / (nxc * nyc)


def kernel(x, y, W, b, gamma_x, beta_x, gamma_y, beta_y):
    row = lambda v: v.reshape(1, EDS)
    out = pl.pallas_call(
        _fused_kernel,
        grid=(NBLK,),
        in_specs=[
            pl.BlockSpec(memory_space=pltpu.MemorySpace.HBM),
            pl.BlockSpec(memory_space=pltpu.MemorySpace.HBM),
            pl.BlockSpec((BLK, EDD), lambda j: (j, 0)),
            pl.BlockSpec((1, BLK), lambda j: (0, j)),
            pl.BlockSpec((1, BLK), lambda j: (0, j)),
            pl.BlockSpec((1, BLK), lambda j: (0, j)),
            pl.BlockSpec((1, BLK), lambda j: (0, j)),
        ],
        out_specs=pl.BlockSpec((B, 1), lambda j: (0, 0)),
        out_shape=jax.ShapeDtypeStruct((B, 1), jnp.float32),
        scratch_shapes=[
            pltpu.VMEM((B, EDD), jnp.float32),
            pltpu.VMEM((B, EDD), jnp.float32),
            pltpu.VMEM((B, 1), jnp.float32),
            pltpu.VMEM((B, 1), jnp.float32),
            pltpu.VMEM((B, 1), jnp.float32),
            pltpu.SemaphoreType.DMA((2,)),
        ],
        compiler_params=pltpu.CompilerParams(
            dimension_semantics=("arbitrary",)),
    )(x, y, W, row(gamma_x), row(beta_x), row(gamma_y), row(beta_y))
    return out.reshape(B)


# speedup vs baseline: 1.2250x; 1.0448x over previous
import jax
import jax.numpy as jnp
from jax import lax
from jax.experimental import pallas as pl
from jax.experimental.pallas import tpu as pltpu

B = 64
EDD = 2048
EDS = 1024
CHUNK = 256
NCHUNK = EDS // CHUNK
BN_EPS = 1e-5
COS_EPS = 1e-8

_DN_T = (((1,), (1,)), ((), ()))
_DN = (((1,), (0,)), ((), ()))


def _fused_kernel(x_ref, y_ref, w_ref, gx_ref, bx_ref, gy_ref, by_ref,
                  out_ref):
    ones_row = jnp.ones((1, B), dtype=jnp.float32)
    ones_col = jnp.ones((CHUNK, 1), dtype=jnp.float32)
    lane = lax.broadcasted_iota(jnp.int32, (B, CHUNK), 1)
    at_block_start = (lane % 4) == 0
    low = jnp.full((B, CHUNK), -2.0, dtype=jnp.float32)

    def bn_tanh(hh, g, bb):
        s1 = lax.dot_general(ones_row, hh, _DN,
                             preferred_element_type=jnp.float32)
        s2 = lax.dot_general(ones_row, hh * hh, _DN,
                             preferred_element_type=jnp.float32)
        mu = s1 * (1.0 / B)
        var = s2 * (1.0 / B) - mu * mu
        scale = lax.rsqrt(var + BN_EPS) * g
        shift = bb - mu * scale
        return jnp.tanh(hh * scale + shift)

    def block_mask(hh):
        a = jnp.maximum(hh, pltpu.roll(hh, CHUNK - 1, 1))
        bm = jnp.maximum(a, pltpu.roll(a, CHUNK - 2, 1))
        c = jnp.where(at_block_start, bm, low)
        c = jnp.maximum(c, pltpu.roll(c, 1, 1))
        bmax = jnp.maximum(c, pltpu.roll(c, 2, 1))
        return jnp.where(hh == bmax, hh, 0.0)

    dot = jnp.zeros((B, 1), dtype=jnp.float32)
    nx = jnp.zeros((B, 1), dtype=jnp.float32)
    ny = jnp.zeros((B, 1), dtype=jnp.float32)
    for k in range(NCHUNK):
        rows = pl.ds(k * CHUNK, CHUNK)
        w = w_ref[rows, :]
        cols = pl.ds(k * CHUNK, CHUNK)
        hx = lax.dot_general(x_ref[...], w, _DN_T,
                             preferred_element_type=jnp.float32)
        hy = lax.dot_general(y_ref[...], w, _DN_T,
                             preferred_element_type=jnp.float32)
        mx = block_mask(bn_tanh(hx, gx_ref[:, cols], bx_ref[:, cols]))
        my = block_mask(bn_tanh(hy, gy_ref[:, cols], by_ref[:, cols]))
        dot += lax.dot_general(mx * my, ones_col, _DN,
                               preferred_element_type=jnp.float32)
        nx += lax.dot_general(mx * mx, ones_col, _DN,
                              preferred_element_type=jnp.float32)
        ny += lax.dot_general(my * my, ones_col, _DN,
                              preferred_element_type=jnp.float32)

    nxc = jnp.maximum(jnp.sqrt(nx), COS_EPS)
    nyc = jnp.maximum(jnp.sqrt(ny), COS_EPS)
    out_ref[...] = dot / (nxc * nyc)


def kernel(x, y, W, b, gamma_x, beta_x, gamma_y, beta_y):
    row = lambda v: v.reshape(1, EDS)
    out = pl.pallas_call(
        _fused_kernel,
        in_specs=[
            pl.BlockSpec((B, EDD), lambda: (0, 0)),
            pl.BlockSpec((B, EDD), lambda: (0, 0)),
            pl.BlockSpec((EDS, EDD), lambda: (0, 0)),
            pl.BlockSpec((1, EDS), lambda: (0, 0)),
            pl.BlockSpec((1, EDS), lambda: (0, 0)),
            pl.BlockSpec((1, EDS), lambda: (0, 0)),
            pl.BlockSpec((1, EDS), lambda: (0, 0)),
        ],
        out_specs=pl.BlockSpec((B, 1), lambda: (0, 0)),
        out_shape=jax.ShapeDtypeStruct((B, 1), jnp.float32),
    )(x, y, W, row(gamma_x), row(beta_x), row(gamma_y), row(beta_y))
    return out.reshape(B)


# speedup vs baseline: 1.3499x; 1.1020x over previous
import jax
import jax.numpy as jnp
from jax import lax
from jax.experimental import pallas as pl
from jax.experimental.pallas import tpu as pltpu

B = 64
EDD = 2048
EDS = 1024
CHUNK = 1024
NCHUNK = EDS // CHUNK
BN_EPS = 1e-5
COS_EPS = 1e-8

_DN_T = (((1,), (1,)), ((), ()))
_DN = (((1,), (0,)), ((), ()))


def _fused_kernel(x_ref, y_ref, w_ref, gx_ref, bx_ref, gy_ref, by_ref,
                  out_ref):
    ones_row = jnp.ones((1, B), dtype=jnp.float32)
    ones_col = jnp.ones((CHUNK, 1), dtype=jnp.float32)
    lane = lax.broadcasted_iota(jnp.int32, (B, CHUNK), 1)
    at_block_start = (lane % 4) == 0
    low = jnp.full((B, CHUNK), -2.0, dtype=jnp.float32)

    def bn_tanh(hh, g, bb):
        s1 = lax.dot_general(ones_row, hh, _DN,
                             preferred_element_type=jnp.float32)
        s2 = lax.dot_general(ones_row, hh * hh, _DN,
                             preferred_element_type=jnp.float32)
        mu = s1 * (1.0 / B)
        var = s2 * (1.0 / B) - mu * mu
        scale = lax.rsqrt(var + BN_EPS) * g
        shift = bb - mu * scale
        return jnp.tanh(hh * scale + shift)

    def block_mask(hh):
        a = jnp.maximum(hh, pltpu.roll(hh, CHUNK - 1, 1))
        bm = jnp.maximum(a, pltpu.roll(a, CHUNK - 2, 1))
        c = jnp.where(at_block_start, bm, low)
        c = jnp.maximum(c, pltpu.roll(c, 1, 1))
        bmax = jnp.maximum(c, pltpu.roll(c, 2, 1))
        return jnp.where(hh == bmax, hh, 0.0)

    dot = jnp.zeros((B, 1), dtype=jnp.float32)
    nx = jnp.zeros((B, 1), dtype=jnp.float32)
    ny = jnp.zeros((B, 1), dtype=jnp.float32)
    for k in range(NCHUNK):
        rows = pl.ds(k * CHUNK, CHUNK)
        w = w_ref[rows, :]
        cols = pl.ds(k * CHUNK, CHUNK)
        hx = lax.dot_general(x_ref[...], w, _DN_T,
                             preferred_element_type=jnp.float32)
        hy = lax.dot_general(y_ref[...], w, _DN_T,
                             preferred_element_type=jnp.float32)
        mx = block_mask(bn_tanh(hx, gx_ref[:, cols], bx_ref[:, cols]))
        my = block_mask(bn_tanh(hy, gy_ref[:, cols], by_ref[:, cols]))
        dot += lax.dot_general(mx * my, ones_col, _DN,
                               preferred_element_type=jnp.float32)
        nx += lax.dot_general(mx * mx, ones_col, _DN,
                              preferred_element_type=jnp.float32)
        ny += lax.dot_general(my * my, ones_col, _DN,
                              preferred_element_type=jnp.float32)

    nxc = jnp.maximum(jnp.sqrt(nx), COS_EPS)
    nyc = jnp.maximum(jnp.sqrt(ny), COS_EPS)
    out_ref[...] = dot / (nxc * nyc)


def kernel(x, y, W, b, gamma_x, beta_x, gamma_y, beta_y):
    row = lambda v: v.reshape(1, EDS)
    out = pl.pallas_call(
        _fused_kernel,
        in_specs=[
            pl.BlockSpec((B, EDD), lambda: (0, 0)),
            pl.BlockSpec((B, EDD), lambda: (0, 0)),
            pl.BlockSpec((EDS, EDD), lambda: (0, 0)),
            pl.BlockSpec((1, EDS), lambda: (0, 0)),
            pl.BlockSpec((1, EDS), lambda: (0, 0)),
            pl.BlockSpec((1, EDS), lambda: (0, 0)),
            pl.BlockSpec((1, EDS), lambda: (0, 0)),
        ],
        out_specs=pl.BlockSpec((B, 1), lambda: (0, 0)),
        out_shape=jax.ShapeDtypeStruct((B, 1), jnp.float32),
    )(x, y, W, row(gamma_x), row(beta_x), row(gamma_y), row(beta_y))
    return out.reshape(B)
